# baseline (speedup 1.0000x reference)
import functools

import jax
import jax.numpy as jnp
from jax import lax
from jax.experimental import pallas as pl
from jax.experimental.pallas import tpu as pltpu
from jax.experimental.pallas import tpu_sc as plsc

N = 10000
D = 128
E = 320000

NC = 2
NS = 16
NW = NC * NS
CHUNK = 128
NCHUNK = 80
NPAIR = NCHUNK // 2
EPT = CHUNK * NCHUNK
EPAD = NW * EPT
SINK = N
AROWS = 10112
ZR = AROWS // NS

RB = 1000
DSIZE = 10240
HCAT = 32


def _make_agg(d):
    mesh = plsc.VectorSubcoreMesh(core_axis_name="c", subcore_axis_name="s",
                                  num_cores=NC, num_subcores=NS)
    out_type = jax.ShapeDtypeStruct((NC, AROWS, d), jnp.float32)
    scratch = [
        pltpu.VMEM((NCHUNK, CHUNK), jnp.int32),
        pltpu.VMEM((CHUNK,), jnp.int32),
        pltpu.VMEM((CHUNK,), jnp.int32),
        pltpu.VMEM((CHUNK,), jnp.int32),
        pltpu.VMEM((CHUNK,), jnp.int32),
        pltpu.VMEM((CHUNK, d), jnp.float32),
        pltpu.VMEM((CHUNK, d), jnp.float32),
        pltpu.VMEM_SHARED((AROWS, d), jnp.float32),
        pltpu.SemaphoreType.DMA,
        pltpu.SemaphoreType.DMA,
    ]

    def body(h, pkm, zrows, acc_out,
             pk_v, srcA, dstA, srcB, dstB, rowsA, rowsB, acc_sh,
             semG, semS):
        c = lax.axis_index("c")
        s = lax.axis_index("s")
        w = c * NS + s

        def issue_g(sv, rows):
            pltpu.async_copy(h.at[sv], rows, semG)

        def wait_g(rows):
            pltpu.make_async_copy(h.at[pl.ds(0, CHUNK)], rows, semG).wait()

        def issue_s(rows, dv):
            pltpu.async_copy(rows, acc_sh.at[dv], semS, add=True)

        def wait_s(rows):
            pltpu.make_async_copy(rows, acc_sh.at[pl.ds(0, CHUNK)], semS).wait()

        def unpack(j, sv, dv):
            for t in range(CHUNK // 16):
                pv = pk_v[j, pl.ds(t * 16, 16)]
                sv[pl.ds(t * 16, 16)] = lax.bitwise_and(pv, 0xFFFF)
                dv[pl.ds(t * 16, 16)] = lax.shift_right_logical(pv, 16)

        pltpu.sync_copy(zrows.at[pl.ds(s * ZR, ZR)], acc_sh.at[pl.ds(s * ZR, ZR)])
        pltpu.sync_copy(pkm.at[w], pk_v)
        plsc.subcore_barrier()

        unpack(0, srcA, dstA)
        issue_g(srcA, rowsA)

        def pair(p, carry):
            wait_g(rowsA)
            issue_s(rowsA, dstA)

            @pl.when(p > 0)
            def _():
                wait_s(rowsB)
            unpack(2 * p + 1, srcB, dstB)
            issue_g(srcB, rowsB)

            wait_g(rowsB)
            issue_s(rowsB, dstB)
            wait_s(rowsA)

            @pl.when(p < NPAIR - 1)
            def _():
                unpack(2 * p + 2, srcA, dstA)
                issue_g(srcA, rowsA)
            return carry

        lax.fori_loop(0, NPAIR, pair, 0)
        wait_s(rowsB)

        plsc.subcore_barrier()
        pltpu.sync_copy(acc_sh.at[pl.ds(s * ZR, ZR)],
                        acc_out.at[c, pl.ds(s * ZR, ZR)])

    return pl.kernel(body, out_type=out_type, mesh=mesh, scratch_types=scratch,
                     compiler_params=pltpu.CompilerParams(needs_layout_passes=False,
                                                          use_tc_tiling_on_sc=False))


def _make_deg():
    mesh = plsc.VectorSubcoreMesh(core_axis_name="c", subcore_axis_name="s",
                                  num_cores=NC, num_subcores=NS)
    out_type = jax.ShapeDtypeStruct((NC, NS, DSIZE), jnp.float32)
    scratch = [
        pltpu.VMEM((NCHUNK, CHUNK), jnp.int32),
        pltpu.VMEM((DSIZE,), jnp.float32),
        pltpu.SemaphoreType.DMA,
    ]

    def body(pkm, deg_out, pk_v, deg_v, sem):
        c = lax.axis_index("c")
        s = lax.axis_index("s")
        w = c * NS + s
        cp = pltpu.async_copy(pkm.at[w], pk_v, sem)

        def zstep(i, carry):
            deg_v[pl.ds(i * 16, 16)] = jnp.zeros((16,), jnp.float32)
            return carry
        lax.fori_loop(0, DSIZE // 16, zstep, 0)
        cp.wait()

        ones = jnp.full((16,), 1.0, jnp.float32)

        def step(j, carry):
            for t in range(CHUNK // 16):
                pv = pk_v[j, pl.ds(t * 16, 16)]
                dvec = lax.shift_right_logical(pv, 16)
                plsc.addupdate_scatter(deg_v, [dvec], ones)
            return carry

        lax.fori_loop(0, NCHUNK, step, 0)
        pltpu.sync_copy(deg_v, deg_out.at[c, s])

    return pl.kernel(body, out_type=out_type, mesh=mesh, scratch_types=scratch,
                     compiler_params=pltpu.CompilerParams(needs_layout_passes=False,
                                                          use_tc_tiling_on_sc=False))


_make_agg = functools.lru_cache(None)(_make_agg)
_make_deg = functools.lru_cache(None)(_make_deg)


def _agg128(*args):
    return _make_agg(D)(*args)


def _agg32(*args):
    return _make_agg(HCAT)(*args)


def _layer_body(a0, a1, deg, x, Wl, Wr, b, out):
    rd = 1.0 / jnp.maximum(deg[...], 1.0)
    mean = (a0[...] + a1[...]) * rd
    h = (jnp.dot(mean, Wl[...], preferred_element_type=jnp.float32)
         + jnp.dot(x[...], Wr[...], preferred_element_type=jnp.float32)
         + b[...])
    out[...] = jnp.maximum(h, 0.0)


def _layer3_body(a0, a1, deg, x, Wl, Wr, b, Wlcat, out, outp):
    rd = 1.0 / jnp.maximum(deg[...], 1.0)
    mean = (a0[...] + a1[...]) * rd
    h = (jnp.dot(mean, Wl[...], preferred_element_type=jnp.float32)
         + jnp.dot(x[...], Wr[...], preferred_element_type=jnp.float32)
         + b[...])
    h = jnp.maximum(h, 0.0)
    out[...] = h
    outp[...] = jnp.dot(h, Wlcat[...], preferred_element_type=jnp.float32)


def _heads_body(a0, a1, deg, h3, Wrcat, bcat, out):
    rd = 1.0 / jnp.maximum(deg[...], 1.0)
    meanp = (a0[...] + a1[...]) * rd
    out[...] = (meanp
                + jnp.dot(h3[...], Wrcat[...], preferred_element_type=jnp.float32)
                + bcat[...])


def _row_spec(cols):
    return pl.BlockSpec((RB, cols), lambda i: (i, 0))


def _full_spec(rows, cols):
    return pl.BlockSpec((rows, cols), lambda i: (0, 0))


def _tc_layer(a0, a1, deg, x, Wl, Wr, b):
    return pl.pallas_call(
        _layer_body,
        grid=(N // RB,),
        in_specs=[_row_spec(D), _row_spec(D), _row_spec(1), _row_spec(D),
                  _full_spec(D, D), _full_spec(D, D), _full_spec(1, D)],
        out_specs=_row_spec(D),
        out_shape=jax.ShapeDtypeStruct((N, D), jnp.float32),
    )(a0, a1, deg, x, Wl, Wr, b)


def _tc_layer3(a0, a1, deg, x, Wl, Wr, b, Wlcat):
    return pl.pallas_call(
        _layer3_body,
        grid=(N // RB,),
        in_specs=[_row_spec(D), _row_spec(D), _row_spec(1), _row_spec(D),
                  _full_spec(D, D), _full_spec(D, D), _full_spec(1, D),
                  _full_spec(D, HCAT)],
        out_specs=[_row_spec(D), _row_spec(HCAT)],
        out_shape=[jax.ShapeDtypeStruct((N, D), jnp.float32),
                   jax.ShapeDtypeStruct((N, HCAT), jnp.float32)],
    )(a0, a1, deg, x, Wl, Wr, b, Wlcat)


def _tc_heads(a0, a1, deg, h3, Wrcat, bcat):
    return pl.pallas_call(
        _heads_body,
        grid=(N // RB,),
        in_specs=[_row_spec(HCAT), _row_spec(HCAT), _row_spec(1), _row_spec(D),
                  _full_spec(D, HCAT), _full_spec(1, HCAT)],
        out_specs=_row_spec(HCAT),
        out_shape=jax.ShapeDtypeStruct((N, HCAT), jnp.float32),
    )(a0, a1, deg, h3, Wrcat, bcat)


def _pad_cat(ws):
    cat = jnp.concatenate(ws, axis=1)
    return jnp.pad(cat, ((0, 0), (0, HCAT - cat.shape[1])))


def kernel(x, edge_index, c1_Wl, c1_Wr, c1_b, c2_Wl, c2_Wr, c2_b,
           c3_Wl, c3_Wr, c3_b, ca_Wl, ca_Wr, ca_b, cs_Wl, cs_Wr, cs_b,
           ce_Wl, ce_Wr, ce_b):
    src = edge_index[0].astype(jnp.int32)
    dst = edge_index[1].astype(jnp.int32)
    pad = EPAD - E
    srcp = jnp.concatenate([src, jnp.zeros((pad,), jnp.int32)])
    dstp = jnp.concatenate([dst, jnp.full((pad,), SINK, jnp.int32)])
    pkm = (srcp | (dstp << 16)).reshape(NW, NCHUNK, CHUNK)
    z128 = jnp.zeros((AROWS, D), jnp.float32)
    z32 = jnp.zeros((AROWS, HCAT), jnp.float32)

    degw = _make_deg()(pkm)
    deg = degw.reshape(NW, DSIZE).sum(axis=0)[:N].reshape(N, 1)
    accx = _agg128(x, pkm, z128)

    h1 = _tc_layer(accx[0, :N], accx[1, :N], deg, x, c1_Wl, c1_Wr,
                   c1_b.reshape(1, D))
    acc1 = _agg128(h1, pkm, z128)
    h2 = _tc_layer(acc1[0, :N], acc1[1, :N], deg, h1, c2_Wl, c2_Wr,
                   c2_b.reshape(1, D))
    acc2 = _agg128(h2, pkm, z128)

    Wlcat = _pad_cat([ca_Wl, cs_Wl, ce_Wl])
    h3, p3 = _tc_layer3(acc2[0, :N], acc2[1, :N], deg, h2, c3_Wl, c3_Wr,
                        c3_b.reshape(1, D), Wlcat)
    accp = _agg32(p3, pkm, z32)

    Wrcat = _pad_cat([ca_Wr, cs_Wr, ce_Wr])
    bcat = jnp.concatenate([ca_b, cs_b, ce_b,
                            jnp.zeros((HCAT - 28,), jnp.float32)]).reshape(1, HCAT)
    outh = _tc_heads(accp[0, :N], accp[1, :N], deg, h3, Wrcat, bcat)
    return outh[:, :21], outh[:, 21:23], outh[:, 23:28]


# speedup vs baseline: 1.0043x; 1.0043x over previous
import functools

import jax
import jax.numpy as jnp
from jax import lax
from jax.experimental import pallas as pl
from jax.experimental.pallas import tpu as pltpu
from jax.experimental.pallas import tpu_sc as plsc

N = 10000
D = 128
E = 320000

NC = 2
NS = 16
NW = NC * NS
CHUNK = 128
NCHUNK = 80
NPAIR = NCHUNK // 2
EPT = CHUNK * NCHUNK
EPAD = NW * EPT
SINK = N
AROWS = 10112
ZR = AROWS // NS

RB = 1000
DSIZE = 10240
HCAT = 32


def _make_agg(d):
    mesh = plsc.VectorSubcoreMesh(core_axis_name="c", subcore_axis_name="s",
                                  num_cores=NC, num_subcores=NS)
    out_type = jax.ShapeDtypeStruct((NC, AROWS, d), jnp.float32)
    scratch = [
        pltpu.VMEM((NCHUNK, CHUNK), jnp.int32),
        pltpu.VMEM((CHUNK,), jnp.int32),
        pltpu.VMEM((CHUNK,), jnp.int32),
        pltpu.VMEM((CHUNK,), jnp.int32),
        pltpu.VMEM((CHUNK,), jnp.int32),
        pltpu.VMEM((CHUNK, d), jnp.float32),
        pltpu.VMEM((CHUNK, d), jnp.float32),
        pltpu.VMEM_SHARED((AROWS, d), jnp.float32),
        pltpu.SemaphoreType.DMA,
    ]

    def body(h, pkm, zrows, acc_out,
             pk_v, srcA, dstA, srcB, dstB, rowsA, rowsB, acc_sh, semG):
        c = lax.axis_index("c")
        s = lax.axis_index("s")
        w = c * NS + s

        def issue_g(sv, rows):
            pltpu.async_copy(h.at[sv], rows, semG)

        def wait_g(rows):
            pltpu.make_async_copy(h.at[pl.ds(0, CHUNK)], rows, semG).wait()

        def scat(rows, dv):
            pltpu.sync_copy(rows, acc_sh.at[dv], add=True)

        def unpack(j, sv, dv):
            for t in range(CHUNK // 16):
                pv = pk_v[j, pl.ds(t * 16, 16)]
                sv[pl.ds(t * 16, 16)] = lax.bitwise_and(pv, 0xFFFF)
                dv[pl.ds(t * 16, 16)] = lax.shift_right_logical(pv, 16)

        pltpu.sync_copy(zrows.at[pl.ds(s * ZR, ZR)], acc_sh.at[pl.ds(s * ZR, ZR)])
        pltpu.sync_copy(pkm.at[w], pk_v)
        plsc.subcore_barrier()

        unpack(0, srcA, dstA)
        issue_g(srcA, rowsA)

        def pair(p, carry):
            unpack(2 * p + 1, srcB, dstB)
            wait_g(rowsA)
            issue_g(srcB, rowsB)
            scat(rowsA, dstA)

            @pl.when(p < NPAIR - 1)
            def _():
                unpack(2 * p + 2, srcA, dstA)
            wait_g(rowsB)

            @pl.when(p < NPAIR - 1)
            def _():
                issue_g(srcA, rowsA)
            scat(rowsB, dstB)
            return carry

        lax.fori_loop(0, NPAIR, pair, 0)

        plsc.subcore_barrier()
        pltpu.sync_copy(acc_sh.at[pl.ds(s * ZR, ZR)],
                        acc_out.at[c, pl.ds(s * ZR, ZR)])

    return pl.kernel(body, out_type=out_type, mesh=mesh, scratch_types=scratch,
                     compiler_params=pltpu.CompilerParams(needs_layout_passes=False,
                                                          use_tc_tiling_on_sc=False))


def _make_deg():
    mesh = plsc.VectorSubcoreMesh(core_axis_name="c", subcore_axis_name="s",
                                  num_cores=NC, num_subcores=NS)
    out_type = jax.ShapeDtypeStruct((NC, NS, DSIZE), jnp.float32)
    scratch = [
        pltpu.VMEM((NCHUNK, CHUNK), jnp.int32),
        pltpu.VMEM((DSIZE,), jnp.float32),
        pltpu.SemaphoreType.DMA,
    ]

    def body(pkm, deg_out, pk_v, deg_v, sem):
        c = lax.axis_index("c")
        s = lax.axis_index("s")
        w = c * NS + s
        cp = pltpu.async_copy(pkm.at[w], pk_v, sem)

        def zstep(i, carry):
            deg_v[pl.ds(i * 16, 16)] = jnp.zeros((16,), jnp.float32)
            return carry
        lax.fori_loop(0, DSIZE // 16, zstep, 0)
        cp.wait()

        ones = jnp.full((16,), 1.0, jnp.float32)

        def step(j, carry):
            for t in range(CHUNK // 16):
                pv = pk_v[j, pl.ds(t * 16, 16)]
                dvec = lax.shift_right_logical(pv, 16)
                plsc.addupdate_scatter(deg_v, [dvec], ones)
            return carry

        lax.fori_loop(0, NCHUNK, step, 0)
        pltpu.sync_copy(deg_v, deg_out.at[c, s])

    return pl.kernel(body, out_type=out_type, mesh=mesh, scratch_types=scratch,
                     compiler_params=pltpu.CompilerParams(needs_layout_passes=False,
                                                          use_tc_tiling_on_sc=False))


_make_agg = functools.lru_cache(None)(_make_agg)
_make_deg = functools.lru_cache(None)(_make_deg)


def _agg128(*args):
    return _make_agg(D)(*args)


def _agg32(*args):
    return _make_agg(HCAT)(*args)


def _layer_body(a0, a1, deg, x, Wl, Wr, b, out):
    rd = 1.0 / jnp.maximum(deg[...], 1.0)
    mean = (a0[...] + a1[...]) * rd
    h = (jnp.dot(mean, Wl[...], preferred_element_type=jnp.float32)
         + jnp.dot(x[...], Wr[...], preferred_element_type=jnp.float32)
         + b[...])
    out[...] = jnp.maximum(h, 0.0)


def _layer3_body(a0, a1, deg, x, Wl, Wr, b, Wlcat, out, outp):
    rd = 1.0 / jnp.maximum(deg[...], 1.0)
    mean = (a0[...] + a1[...]) * rd
    h = (jnp.dot(mean, Wl[...], preferred_element_type=jnp.float32)
         + jnp.dot(x[...], Wr[...], preferred_element_type=jnp.float32)
         + b[...])
    h = jnp.maximum(h, 0.0)
    out[...] = h
    outp[...] = jnp.dot(h, Wlcat[...], preferred_element_type=jnp.float32)


def _heads_body(a0, a1, deg, h3, Wrcat, bcat, out):
    rd = 1.0 / jnp.maximum(deg[...], 1.0)
    meanp = (a0[...] + a1[...]) * rd
    out[...] = (meanp
                + jnp.dot(h3[...], Wrcat[...], preferred_element_type=jnp.float32)
                + bcat[...])


def _row_spec(cols):
    return pl.BlockSpec((RB, cols), lambda i: (i, 0))


def _full_spec(rows, cols):
    return pl.BlockSpec((rows, cols), lambda i: (0, 0))


def _tc_layer(a0, a1, deg, x, Wl, Wr, b):
    return pl.pallas_call(
        _layer_body,
        grid=(N // RB,),
        in_specs=[_row_spec(D), _row_spec(D), _row_spec(1), _row_spec(D),
                  _full_spec(D, D), _full_spec(D, D), _full_spec(1, D)],
        out_specs=_row_spec(D),
        out_shape=jax.ShapeDtypeStruct((N, D), jnp.float32),
    )(a0, a1, deg, x, Wl, Wr, b)


def _tc_layer3(a0, a1, deg, x, Wl, Wr, b, Wlcat):
    return pl.pallas_call(
        _layer3_body,
        grid=(N // RB,),
        in_specs=[_row_spec(D), _row_spec(D), _row_spec(1), _row_spec(D),
                  _full_spec(D, D), _full_spec(D, D), _full_spec(1, D),
                  _full_spec(D, HCAT)],
        out_specs=[_row_spec(D), _row_spec(HCAT)],
        out_shape=[jax.ShapeDtypeStruct((N, D), jnp.float32),
                   jax.ShapeDtypeStruct((N, HCAT), jnp.float32)],
    )(a0, a1, deg, x, Wl, Wr, b, Wlcat)


def _tc_heads(a0, a1, deg, h3, Wrcat, bcat):
    return pl.pallas_call(
        _heads_body,
        grid=(N // RB,),
        in_specs=[_row_spec(HCAT), _row_spec(HCAT), _row_spec(1), _row_spec(D),
                  _full_spec(D, HCAT), _full_spec(1, HCAT)],
        out_specs=_row_spec(HCAT),
        out_shape=jax.ShapeDtypeStruct((N, HCAT), jnp.float32),
    )(a0, a1, deg, h3, Wrcat, bcat)


def _pad_cat(ws):
    cat = jnp.concatenate(ws, axis=1)
    return jnp.pad(cat, ((0, 0), (0, HCAT - cat.shape[1])))


def kernel(x, edge_index, c1_Wl, c1_Wr, c1_b, c2_Wl, c2_Wr, c2_b,
           c3_Wl, c3_Wr, c3_b, ca_Wl, ca_Wr, ca_b, cs_Wl, cs_Wr, cs_b,
           ce_Wl, ce_Wr, ce_b):
    src = edge_index[0].astype(jnp.int32)
    dst = edge_index[1].astype(jnp.int32)
    pad = EPAD - E
    srcp = jnp.concatenate([src, jnp.zeros((pad,), jnp.int32)])
    dstp = jnp.concatenate([dst, jnp.full((pad,), SINK, jnp.int32)])
    pkm = (srcp | (dstp << 16)).reshape(NW, NCHUNK, CHUNK)
    z128 = jnp.zeros((AROWS, D), jnp.float32)
    z32 = jnp.zeros((AROWS, HCAT), jnp.float32)

    degw = _make_deg()(pkm)
    deg = degw.reshape(NW, DSIZE).sum(axis=0)[:N].reshape(N, 1)
    accx = _agg128(x, pkm, z128)

    h1 = _tc_layer(accx[0, :N], accx[1, :N], deg, x, c1_Wl, c1_Wr,
                   c1_b.reshape(1, D))
    acc1 = _agg128(h1, pkm, z128)
    h2 = _tc_layer(acc1[0, :N], acc1[1, :N], deg, h1, c2_Wl, c2_Wr,
                   c2_b.reshape(1, D))
    acc2 = _agg128(h2, pkm, z128)

    Wlcat = _pad_cat([ca_Wl, cs_Wl, ce_Wl])
    h3, p3 = _tc_layer3(acc2[0, :N], acc2[1, :N], deg, h2, c3_Wl, c3_Wr,
                        c3_b.reshape(1, D), Wlcat)
    accp = _agg32(p3, pkm, z32)

    Wrcat = _pad_cat([ca_Wr, cs_Wr, ce_Wr])
    bcat = jnp.concatenate([ca_b, cs_b, ce_b,
                            jnp.zeros((HCAT - 28,), jnp.float32)]).reshape(1, HCAT)
    outh = _tc_heads(accp[0, :N], accp[1, :N], deg, h3, Wrcat, bcat)
    return outh[:, :21], outh[:, 21:23], outh[:, 23:28]


# speedup vs baseline: 1.0110x; 1.0067x over previous
import functools

import jax
import jax.numpy as jnp
from jax import lax
from jax.experimental import pallas as pl
from jax.experimental.pallas import tpu as pltpu
from jax.experimental.pallas import tpu_sc as plsc

N = 10000
D = 128
E = 320000

NC = 2
NS = 16
NW = NC * NS
CHUNK = 128
NCHUNK = 80
NPAIR = NCHUNK // 2
EPT = CHUNK * NCHUNK
EPAD = NW * EPT
SINK = N
AROWS = 10112
ZR = AROWS // NS

RB = 1000
DSIZE = 10240
HCAT = 32


HB = NCHUNK // 2
HPAIR = HB // 2


def _make_agg(d):
    mesh = plsc.VectorSubcoreMesh(core_axis_name="c", subcore_axis_name="s",
                                  num_cores=NC, num_subcores=NS)
    out_type = jax.ShapeDtypeStruct((NC, AROWS, d), jnp.float32)
    scratch = [
        pltpu.VMEM((HB, CHUNK), jnp.int32),
        pltpu.VMEM((HB, CHUNK), jnp.int32),
        pltpu.VMEM((CHUNK, d), jnp.float32),
        pltpu.VMEM((CHUNK, d), jnp.float32),
        pltpu.VMEM_SHARED((AROWS, d), jnp.float32),
        pltpu.SemaphoreType.DMA,
    ]

    def body(h, srcm, dstm, zrows, acc_out,
             src_v, dst_v, rowsA, rowsB, acc_sh, semG):
        c = lax.axis_index("c")
        s = lax.axis_index("s")
        w = c * NS + s

        def issue_g(sv, rows):
            pltpu.async_copy(h.at[sv], rows, semG)

        def wait_g(rows):
            pltpu.make_async_copy(h.at[pl.ds(0, CHUNK)], rows, semG).wait()

        def scat(rows, dv):
            pltpu.sync_copy(rows, acc_sh.at[dv], add=True)

        pltpu.sync_copy(zrows.at[pl.ds(s * ZR, ZR)], acc_sh.at[pl.ds(s * ZR, ZR)])
        plsc.subcore_barrier()

        for half in range(2):
            pltpu.sync_copy(srcm.at[w, pl.ds(half * HB, HB)], src_v)
            pltpu.sync_copy(dstm.at[w, pl.ds(half * HB, HB)], dst_v)
            issue_g(src_v.at[0], rowsA)

            def pair(p, carry):
                wait_g(rowsA)
                issue_g(src_v.at[2 * p + 1], rowsB)
                scat(rowsA, dst_v.at[2 * p])
                wait_g(rowsB)

                @pl.when(p < HPAIR - 1)
                def _():
                    issue_g(src_v.at[2 * p + 2], rowsA)
                scat(rowsB, dst_v.at[2 * p + 1])
                return carry

            lax.fori_loop(0, HPAIR, pair, 0)

        plsc.subcore_barrier()
        pltpu.sync_copy(acc_sh.at[pl.ds(s * ZR, ZR)],
                        acc_out.at[c, pl.ds(s * ZR, ZR)])

    return pl.kernel(body, out_type=out_type, mesh=mesh, scratch_types=scratch,
                     compiler_params=pltpu.CompilerParams(needs_layout_passes=False,
                                                          use_tc_tiling_on_sc=False))


def _make_deg():
    mesh = plsc.VectorSubcoreMesh(core_axis_name="c", subcore_axis_name="s",
                                  num_cores=NC, num_subcores=NS)
    out_type = jax.ShapeDtypeStruct((NC, NS, DSIZE), jnp.float32)
    scratch = [
        pltpu.VMEM((NCHUNK, CHUNK), jnp.int32),
        pltpu.VMEM((DSIZE,), jnp.float32),
        pltpu.SemaphoreType.DMA,
    ]

    def body(pkm, deg_out, pk_v, deg_v, sem):
        c = lax.axis_index("c")
        s = lax.axis_index("s")
        w = c * NS + s
        cp = pltpu.async_copy(pkm.at[w], pk_v, sem)

        def zstep(i, carry):
            deg_v[pl.ds(i * 16, 16)] = jnp.zeros((16,), jnp.float32)
            return carry
        lax.fori_loop(0, DSIZE // 16, zstep, 0)
        cp.wait()

        ones = jnp.full((16,), 1.0, jnp.float32)

        def step(j, carry):
            for t in range(CHUNK // 16):
                pv = pk_v[j, pl.ds(t * 16, 16)]
                dvec = lax.shift_right_logical(pv, 16)
                plsc.addupdate_scatter(deg_v, [dvec], ones)
            return carry

        lax.fori_loop(0, NCHUNK, step, 0)
        pltpu.sync_copy(deg_v, deg_out.at[c, s])

    return pl.kernel(body, out_type=out_type, mesh=mesh, scratch_types=scratch,
                     compiler_params=pltpu.CompilerParams(needs_layout_passes=False,
                                                          use_tc_tiling_on_sc=False))


_make_agg = functools.lru_cache(None)(_make_agg)
_make_deg = functools.lru_cache(None)(_make_deg)


def _agg128(*args):
    return _make_agg(D)(*args)


def _agg32(*args):
    return _make_agg(HCAT)(*args)


def _layer_body(a0, a1, deg, x, Wl, Wr, b, out):
    rd = 1.0 / jnp.maximum(deg[...], 1.0)
    mean = (a0[...] + a1[...]) * rd
    h = (jnp.dot(mean, Wl[...], preferred_element_type=jnp.float32)
         + jnp.dot(x[...], Wr[...], preferred_element_type=jnp.float32)
         + b[...])
    out[...] = jnp.maximum(h, 0.0)


def _layer3_body(a0, a1, deg, x, Wl, Wr, b, Wlcat, out, outp):
    rd = 1.0 / jnp.maximum(deg[...], 1.0)
    mean = (a0[...] + a1[...]) * rd
    h = (jnp.dot(mean, Wl[...], preferred_element_type=jnp.float32)
         + jnp.dot(x[...], Wr[...], preferred_element_type=jnp.float32)
         + b[...])
    h = jnp.maximum(h, 0.0)
    out[...] = h
    outp[...] = jnp.dot(h, Wlcat[...], preferred_element_type=jnp.float32)


def _heads_body(a0, a1, deg, h3, Wrcat, bcat, out):
    rd = 1.0 / jnp.maximum(deg[...], 1.0)
    meanp = (a0[...] + a1[...]) * rd
    out[...] = (meanp
                + jnp.dot(h3[...], Wrcat[...], preferred_element_type=jnp.float32)
                + bcat[...])


def _row_spec(cols):
    return pl.BlockSpec((RB, cols), lambda i: (i, 0))


def _full_spec(rows, cols):
    return pl.BlockSpec((rows, cols), lambda i: (0, 0))


def _tc_layer(a0, a1, deg, x, Wl, Wr, b):
    return pl.pallas_call(
        _layer_body,
        grid=(N // RB,),
        in_specs=[_row_spec(D), _row_spec(D), _row_spec(1), _row_spec(D),
                  _full_spec(D, D), _full_spec(D, D), _full_spec(1, D)],
        out_specs=_row_spec(D),
        out_shape=jax.ShapeDtypeStruct((N, D), jnp.float32),
    )(a0, a1, deg, x, Wl, Wr, b)


def _tc_layer3(a0, a1, deg, x, Wl, Wr, b, Wlcat):
    return pl.pallas_call(
        _layer3_body,
        grid=(N // RB,),
        in_specs=[_row_spec(D), _row_spec(D), _row_spec(1), _row_spec(D),
                  _full_spec(D, D), _full_spec(D, D), _full_spec(1, D),
                  _full_spec(D, HCAT)],
        out_specs=[_row_spec(D), _row_spec(HCAT)],
        out_shape=[jax.ShapeDtypeStruct((N, D), jnp.float32),
                   jax.ShapeDtypeStruct((N, HCAT), jnp.float32)],
    )(a0, a1, deg, x, Wl, Wr, b, Wlcat)


def _tc_heads(a0, a1, deg, h3, Wrcat, bcat):
    return pl.pallas_call(
        _heads_body,
        grid=(N // RB,),
        in_specs=[_row_spec(HCAT), _row_spec(HCAT), _row_spec(1), _row_spec(D),
                  _full_spec(D, HCAT), _full_spec(1, HCAT)],
        out_specs=_row_spec(HCAT),
        out_shape=jax.ShapeDtypeStruct((N, HCAT), jnp.float32),
    )(a0, a1, deg, h3, Wrcat, bcat)


def _pad_cat(ws):
    cat = jnp.concatenate(ws, axis=1)
    return jnp.pad(cat, ((0, 0), (0, HCAT - cat.shape[1])))


def kernel(x, edge_index, c1_Wl, c1_Wr, c1_b, c2_Wl, c2_Wr, c2_b,
           c3_Wl, c3_Wr, c3_b, ca_Wl, ca_Wr, ca_b, cs_Wl, cs_Wr, cs_b,
           ce_Wl, ce_Wr, ce_b):
    src = edge_index[0].astype(jnp.int32)
    dst = edge_index[1].astype(jnp.int32)
    pad = EPAD - E
    srcp = jnp.concatenate([src, jnp.zeros((pad,), jnp.int32)])
    dstp = jnp.concatenate([dst, jnp.full((pad,), SINK, jnp.int32)])
    pkm = (srcp | (dstp << 16)).reshape(NW, NCHUNK, CHUNK)
    srcm = srcp.reshape(NW, NCHUNK, CHUNK)
    dstm = dstp.reshape(NW, NCHUNK, CHUNK)
    z128 = jnp.zeros((AROWS, D), jnp.float32)
    z32 = jnp.zeros((AROWS, HCAT), jnp.float32)

    degw = _make_deg()(pkm)
    deg = degw.reshape(NW, DSIZE).sum(axis=0)[:N].reshape(N, 1)
    accx = _agg128(x, srcm, dstm, z128)

    h1 = _tc_layer(accx[0, :N], accx[1, :N], deg, x, c1_Wl, c1_Wr,
                   c1_b.reshape(1, D))
    acc1 = _agg128(h1, srcm, dstm, z128)
    h2 = _tc_layer(acc1[0, :N], acc1[1, :N], deg, h1, c2_Wl, c2_Wr,
                   c2_b.reshape(1, D))
    acc2 = _agg128(h2, srcm, dstm, z128)

    Wlcat = _pad_cat([ca_Wl, cs_Wl, ce_Wl])
    h3, p3 = _tc_layer3(acc2[0, :N], acc2[1, :N], deg, h2, c3_Wl, c3_Wr,
                        c3_b.reshape(1, D), Wlcat)
    accp = _agg32(p3, srcm, dstm, z32)

    Wrcat = _pad_cat([ca_Wr, cs_Wr, ce_Wr])
    bcat = jnp.concatenate([ca_b, cs_b, ce_b,
                            jnp.zeros((HCAT - 28,), jnp.float32)]).reshape(1, HCAT)
    outh = _tc_heads(accp[0, :N], accp[1, :N], deg, h3, Wrcat, bcat)
    return outh[:, :21], outh[:, 21:23], outh[:, 23:28]


# speedup vs baseline: 1.4280x; 1.4125x over previous
import functools

import jax
import jax.numpy as jnp
from jax import lax
from jax.experimental import pallas as pl
from jax.experimental.pallas import tpu as pltpu
from jax.experimental.pallas import tpu_sc as plsc

N = 10000
D = 128
E = 320000

NC = 2
NS = 16
NW = NC * NS
CHUNK = 128
NCHUNK = 79
EPT = CHUNK * NCHUNK
EPAD = NW * EPT
SINK = N
AROWS = 10112
ZR = AROWS // NS

RB = 1000
DSIZE = 10240
HCAT = 32


def _make_agg(d, with_deg, chunk):
    nchunk = EPT // chunk
    mesh = plsc.VectorSubcoreMesh(core_axis_name="c", subcore_axis_name="s",
                                  num_cores=NC, num_subcores=NS)
    out_type = [jax.ShapeDtypeStruct((NC, AROWS, d), jnp.float32)]
    scratch = [
        pltpu.VMEM((nchunk, chunk), jnp.int32),
        pltpu.VMEM((nchunk, chunk), jnp.int32),
        pltpu.VMEM((chunk, d), jnp.float32),
        pltpu.VMEM_SHARED((AROWS, d), jnp.float32),
        pltpu.SemaphoreType.DMA,
    ]
    if with_deg:
        out_type.append(jax.ShapeDtypeStruct((NC, NS, DSIZE), jnp.float32))
        scratch += [
            pltpu.VMEM((DSIZE,), jnp.float32),
        ]

    def body(h, srcm, dstm, zrows, *rest):
        if with_deg:
            (acc_out, deg_out,
             src_v, dst_v, rows_v, acc_sh, sem, deg_v) = rest
        else:
            acc_out, src_v, dst_v, rows_v, acc_sh, sem = rest
        c = lax.axis_index("c")
        s = lax.axis_index("s")
        w = c * NS + s
        pltpu.sync_copy(srcm.at[w], src_v)
        pltpu.sync_copy(dstm.at[w], dst_v)
        pltpu.sync_copy(zrows.at[pl.ds(s * ZR, ZR)], acc_sh.at[pl.ds(s * ZR, ZR)])
        if with_deg:
            def zstep(i, carry):
                deg_v[pl.ds(i * 16, 16)] = jnp.zeros((16,), jnp.float32)
                return carry
            lax.fori_loop(0, DSIZE // 16, zstep, 0)
        plsc.subcore_barrier()

        ones = jnp.full((16,), 1.0, jnp.float32)

        def step(j, carry):
            pltpu.async_copy(h.at[src_v.at[j]], rows_v, sem).wait()
            pltpu.sync_copy(rows_v, acc_sh.at[dst_v.at[j]], add=True)
            if with_deg:
                for k in range(chunk // 16):
                    dvec = dst_v[j, pl.ds(k * 16, 16)]
                    plsc.addupdate_scatter(deg_v, [dvec], ones)
            return carry

        lax.fori_loop(0, nchunk, step, 0)
        plsc.subcore_barrier()
        if with_deg:
            pltpu.sync_copy(deg_v, deg_out.at[c, s])
        pltpu.sync_copy(acc_sh.at[pl.ds(s * ZR, ZR)],
                        acc_out.at[c, pl.ds(s * ZR, ZR)])

    if not with_deg:
        out_type = out_type[0]
    return pl.kernel(body, out_type=out_type, mesh=mesh, scratch_types=scratch,
                     compiler_params=pltpu.CompilerParams(needs_layout_passes=False,
                                                          use_tc_tiling_on_sc=False))


_make_agg = functools.lru_cache(None)(_make_agg)


def _agg_deg(*args):
    return _make_agg(D, True, CHUNK)(*args)


def _agg128(*args):
    return _make_agg(D, False, CHUNK)(*args)


def _agg32(*args):
    return _make_agg(HCAT, False, CHUNK)(*args)


def _layer_body(a0, a1, deg, x, Wl, Wr, b, out):
    rd = 1.0 / jnp.maximum(deg[...], 1.0)
    mean = (a0[...] + a1[...]) * rd
    h = (jnp.dot(mean, Wl[...], preferred_element_type=jnp.float32)
         + jnp.dot(x[...], Wr[...], preferred_element_type=jnp.float32)
         + b[...])
    out[...] = jnp.maximum(h, 0.0)


def _layer3_body(a0, a1, deg, x, Wl, Wr, b, Wlcat, out, outp):
    rd = 1.0 / jnp.maximum(deg[...], 1.0)
    mean = (a0[...] + a1[...]) * rd
    h = (jnp.dot(mean, Wl[...], preferred_element_type=jnp.float32)
         + jnp.dot(x[...], Wr[...], preferred_element_type=jnp.float32)
         + b[...])
    h = jnp.maximum(h, 0.0)
    out[...] = h
    outp[...] = jnp.dot(h, Wlcat[...], preferred_element_type=jnp.float32)


def _heads_body(a0, a1, deg, h3, Wrcat, bcat, out):
    rd = 1.0 / jnp.maximum(deg[...], 1.0)
    meanp = (a0[...] + a1[...]) * rd
    out[...] = (meanp
                + jnp.dot(h3[...], Wrcat[...], preferred_element_type=jnp.float32)
                + bcat[...])


def _row_spec(cols):
    return pl.BlockSpec((RB, cols), lambda i: (i, 0))


def _full_spec(rows, cols):
    return pl.BlockSpec((rows, cols), lambda i: (0, 0))


def _tc_layer(a0, a1, deg, x, Wl, Wr, b):
    return pl.pallas_call(
        _layer_body,
        grid=(N // RB,),
        in_specs=[_row_spec(D), _row_spec(D), _row_spec(1), _row_spec(D),
                  _full_spec(D, D), _full_spec(D, D), _full_spec(1, D)],
        out_specs=_row_spec(D),
        out_shape=jax.ShapeDtypeStruct((N, D), jnp.float32),
    )(a0, a1, deg, x, Wl, Wr, b)


def _tc_layer3(a0, a1, deg, x, Wl, Wr, b, Wlcat):
    return pl.pallas_call(
        _layer3_body,
        grid=(N // RB,),
        in_specs=[_row_spec(D), _row_spec(D), _row_spec(1), _row_spec(D),
                  _full_spec(D, D), _full_spec(D, D), _full_spec(1, D),
                  _full_spec(D, HCAT)],
        out_specs=[_row_spec(D), _row_spec(HCAT)],
        out_shape=[jax.ShapeDtypeStruct((N, D), jnp.float32),
                   jax.ShapeDtypeStruct((N, HCAT), jnp.float32)],
    )(a0, a1, deg, x, Wl, Wr, b, Wlcat)


def _tc_heads(a0, a1, deg, h3, Wrcat, bcat):
    return pl.pallas_call(
        _heads_body,
        grid=(N // RB,),
        in_specs=[_row_spec(HCAT), _row_spec(HCAT), _row_spec(1), _row_spec(D),
                  _full_spec(D, HCAT), _full_spec(1, HCAT)],
        out_specs=_row_spec(HCAT),
        out_shape=jax.ShapeDtypeStruct((N, HCAT), jnp.float32),
    )(a0, a1, deg, h3, Wrcat, bcat)


def _pad_cat(ws):
    cat = jnp.concatenate(ws, axis=1)
    return jnp.pad(cat, ((0, 0), (0, HCAT - cat.shape[1])))


def kernel(x, edge_index, c1_Wl, c1_Wr, c1_b, c2_Wl, c2_Wr, c2_b,
           c3_Wl, c3_Wr, c3_b, ca_Wl, ca_Wr, ca_b, cs_Wl, cs_Wr, cs_b,
           ce_Wl, ce_Wr, ce_b):
    src = edge_index[0].astype(jnp.int32)
    dst = edge_index[1].astype(jnp.int32)
    pad = EPAD - E
    srcm = jnp.concatenate([src, jnp.zeros((pad,), jnp.int32)]).reshape(NW, NCHUNK, CHUNK)
    sinks = SINK + (jnp.arange(pad, dtype=jnp.int32) % (AROWS - N))
    dstm = jnp.concatenate([dst, sinks]).reshape(NW, NCHUNK, CHUNK)
    z128 = jnp.zeros((AROWS, D), jnp.float32)
    z32 = jnp.zeros((AROWS, HCAT), jnp.float32)

    accx, degw = _agg_deg(x, srcm, dstm, z128)
    deg = degw.reshape(NW, DSIZE).sum(axis=0)[:N].reshape(N, 1)

    h1 = _tc_layer(accx[0, :N], accx[1, :N], deg, x, c1_Wl, c1_Wr,
                   c1_b.reshape(1, D))
    acc1 = _agg128(h1, srcm, dstm, z128)
    h2 = _tc_layer(acc1[0, :N], acc1[1, :N], deg, h1, c2_Wl, c2_Wr,
                   c2_b.reshape(1, D))
    acc2 = _agg128(h2, srcm, dstm, z128)

    Wlcat = _pad_cat([ca_Wl, cs_Wl, ce_Wl])
    h3, p3 = _tc_layer3(acc2[0, :N], acc2[1, :N], deg, h2, c3_Wl, c3_Wr,
                        c3_b.reshape(1, D), Wlcat)
    accp = _agg32(p3, srcm, dstm, z32)

    Wrcat = _pad_cat([ca_Wr, cs_Wr, ce_Wr])
    bcat = jnp.concatenate([ca_b, cs_b, ce_b,
                            jnp.zeros((HCAT - 28,), jnp.float32)]).reshape(1, HCAT)
    outh = _tc_heads(accp[0, :N], accp[1, :N], deg, h3, Wrcat, bcat)
    return outh[:, :21], outh[:, 21:23], outh[:, 23:28]


# speedup vs baseline: 1.8509x; 1.2961x over previous
import functools

import jax
import jax.numpy as jnp
from jax import lax
from jax.experimental import pallas as pl
from jax.experimental.pallas import tpu as pltpu
from jax.experimental.pallas import tpu_sc as plsc

N = 10000
D = 128
E = 320000

NC = 2
NS = 16
NW = NC * NS
CHUNK = 128
NCH0 = 102
NCH1 = 55
NCHMAX = 102
CH0 = NS * NCH0
CH_TOT = NS * (NCH0 + NCH1)
ROWS_TOT = 2560
EPAD = ROWS_TOT * CHUNK
SINK = N
AROWS = 10112
ZR = AROWS // NS

RB = 1000
DSIZE = 10240
HCAT = 32


def _chunk_bounds(c, s):
    b = jnp.where(c == 0, NCH0, NCH1)
    off = jnp.where(c == 0, s * NCH0, CH0 + s * NCH1)
    return b, off


def _make_agg(d):
    mesh = plsc.VectorSubcoreMesh(core_axis_name="c", subcore_axis_name="s",
                                  num_cores=NC, num_subcores=NS)
    out_type = jax.ShapeDtypeStruct((NC, AROWS, d), jnp.float32)
    scratch = [
        pltpu.VMEM((NCHMAX, CHUNK), jnp.int32),
        pltpu.VMEM((NCHMAX, CHUNK), jnp.int32),
        pltpu.VMEM((CHUNK, d), jnp.float32),
        pltpu.VMEM_SHARED((AROWS, d), jnp.float32),
        pltpu.SemaphoreType.DMA,
    ]

    def body(h, srcm, dstm, zrows, acc_out,
             src_v, dst_v, rows_v, acc_sh, sem):
        c = lax.axis_index("c")
        s = lax.axis_index("s")
        b, off = _chunk_bounds(c, s)
        pltpu.sync_copy(srcm.at[pl.ds(off, NCHMAX)], src_v)
        pltpu.sync_copy(dstm.at[pl.ds(off, NCHMAX)], dst_v)
        pltpu.sync_copy(zrows.at[pl.ds(s * ZR, ZR)], acc_sh.at[pl.ds(s * ZR, ZR)])
        plsc.subcore_barrier()

        def step(j, carry):
            @pl.when(j < b)
            def _():
                pltpu.async_copy(h.at[src_v.at[j]], rows_v, sem).wait()
                pltpu.sync_copy(rows_v, acc_sh.at[dst_v.at[j]], add=True)
            return carry

        lax.fori_loop(0, NCHMAX, step, 0)
        plsc.subcore_barrier()
        pltpu.sync_copy(acc_sh.at[pl.ds(s * ZR, ZR)],
                        acc_out.at[c, pl.ds(s * ZR, ZR)])

    return pl.kernel(body, out_type=out_type, mesh=mesh, scratch_types=scratch,
                     compiler_params=pltpu.CompilerParams(needs_layout_passes=False,
                                                          use_tc_tiling_on_sc=False))


def _make_deg():
    mesh = plsc.VectorSubcoreMesh(core_axis_name="c", subcore_axis_name="s",
                                  num_cores=NC, num_subcores=NS)
    out_type = jax.ShapeDtypeStruct((NC, NS, DSIZE), jnp.float32)
    scratch = [
        pltpu.VMEM((NCHMAX, CHUNK), jnp.int32),
        pltpu.VMEM((DSIZE,), jnp.float32),
        pltpu.SemaphoreType.DMA,
    ]

    def body(dstm, deg_out, dst_v, deg_v, sem):
        c = lax.axis_index("c")
        s = lax.axis_index("s")
        b, off = _chunk_bounds(c, s)
        cp = pltpu.async_copy(dstm.at[pl.ds(off, NCHMAX)], dst_v, sem)

        def zstep(i, carry):
            deg_v[pl.ds(i * 16, 16)] = jnp.zeros((16,), jnp.float32)
            return carry
        lax.fori_loop(0, DSIZE // 16, zstep, 0)
        cp.wait()

        ones = jnp.full((16,), 1.0, jnp.float32)

        def step(j, carry):
            @pl.when(j < b)
            def _():
                for t in range(CHUNK // 16):
                    dvec = dst_v[j, pl.ds(t * 16, 16)]
                    plsc.addupdate_scatter(deg_v, [dvec], ones)
            return carry

        lax.fori_loop(0, NCHMAX, step, 0)
        pltpu.sync_copy(deg_v, deg_out.at[c, s])

    return pl.kernel(body, out_type=out_type, mesh=mesh, scratch_types=scratch,
                     compiler_params=pltpu.CompilerParams(needs_layout_passes=False,
                                                          use_tc_tiling_on_sc=False))


_make_agg = functools.lru_cache(None)(_make_agg)
_make_deg = functools.lru_cache(None)(_make_deg)


def _agg128(*args):
    return _make_agg(D)(*args)


def _agg32(*args):
    return _make_agg(HCAT)(*args)


def _layer_body(a0, a1, deg, x, Wl, Wr, b, out):
    rd = 1.0 / jnp.maximum(deg[...], 1.0)
    mean = (a0[...] + a1[...]) * rd
    h = (jnp.dot(mean, Wl[...], preferred_element_type=jnp.float32)
         + jnp.dot(x[...], Wr[...], preferred_element_type=jnp.float32)
         + b[...])
    out[...] = jnp.maximum(h, 0.0)


def _layer3_body(a0, a1, deg, x, Wl, Wr, b, Wlcat, out, outp):
    rd = 1.0 / jnp.maximum(deg[...], 1.0)
    mean = (a0[...] + a1[...]) * rd
    h = (jnp.dot(mean, Wl[...], preferred_element_type=jnp.float32)
         + jnp.dot(x[...], Wr[...], preferred_element_type=jnp.float32)
         + b[...])
    h = jnp.maximum(h, 0.0)
    out[...] = h
    outp[...] = jnp.dot(h, Wlcat[...], preferred_element_type=jnp.float32)


def _heads_body(a0, a1, deg, h3, Wrcat, bcat, out):
    rd = 1.0 / jnp.maximum(deg[...], 1.0)
    meanp = (a0[...] + a1[...]) * rd
    out[...] = (meanp
                + jnp.dot(h3[...], Wrcat[...], preferred_element_type=jnp.float32)
                + bcat[...])


def _row_spec(cols):
    return pl.BlockSpec((RB, cols), lambda i: (i, 0))


def _full_spec(rows, cols):
    return pl.BlockSpec((rows, cols), lambda i: (0, 0))


def _tc_layer(a0, a1, deg, x, Wl, Wr, b):
    return pl.pallas_call(
        _layer_body,
        grid=(N // RB,),
        in_specs=[_row_spec(D), _row_spec(D), _row_spec(1), _row_spec(D),
                  _full_spec(D, D), _full_spec(D, D), _full_spec(1, D)],
        out_specs=_row_spec(D),
        out_shape=jax.ShapeDtypeStruct((N, D), jnp.float32),
    )(a0, a1, deg, x, Wl, Wr, b)


def _tc_layer3(a0, a1, deg, x, Wl, Wr, b, Wlcat):
    return pl.pallas_call(
        _layer3_body,
        grid=(N // RB,),
        in_specs=[_row_spec(D), _row_spec(D), _row_spec(1), _row_spec(D),
                  _full_spec(D, D), _full_spec(D, D), _full_spec(1, D),
                  _full_spec(D, HCAT)],
        out_specs=[_row_spec(D), _row_spec(HCAT)],
        out_shape=[jax.ShapeDtypeStruct((N, D), jnp.float32),
                   jax.ShapeDtypeStruct((N, HCAT), jnp.float32)],
    )(a0, a1, deg, x, Wl, Wr, b, Wlcat)


def _tc_heads(a0, a1, deg, h3, Wrcat, bcat):
    return pl.pallas_call(
        _heads_body,
        grid=(N // RB,),
        in_specs=[_row_spec(HCAT), _row_spec(HCAT), _row_spec(1), _row_spec(D),
                  _full_spec(D, HCAT), _full_spec(1, HCAT)],
        out_specs=_row_spec(HCAT),
        out_shape=jax.ShapeDtypeStruct((N, HCAT), jnp.float32),
    )(a0, a1, deg, h3, Wrcat, bcat)


def _pad_cat(ws):
    cat = jnp.concatenate(ws, axis=1)
    return jnp.pad(cat, ((0, 0), (0, HCAT - cat.shape[1])))


def kernel(x, edge_index, c1_Wl, c1_Wr, c1_b, c2_Wl, c2_Wr, c2_b,
           c3_Wl, c3_Wr, c3_b, ca_Wl, ca_Wr, ca_b, cs_Wl, cs_Wr, cs_b,
           ce_Wl, ce_Wr, ce_b):
    src = edge_index[0].astype(jnp.int32)
    dst = edge_index[1].astype(jnp.int32)
    pad = EPAD - E
    srcm = jnp.concatenate([src, jnp.zeros((pad,), jnp.int32)]).reshape(ROWS_TOT, CHUNK)
    sinks = SINK + (jnp.arange(pad, dtype=jnp.int32) % (AROWS - N))
    dstm = jnp.concatenate([dst, sinks]).reshape(ROWS_TOT, CHUNK)
    z128 = jnp.zeros((AROWS, D), jnp.float32)
    z32 = jnp.zeros((AROWS, HCAT), jnp.float32)

    degw = _make_deg()(dstm)
    deg = degw.reshape(NW, DSIZE).sum(axis=0)[:N].reshape(N, 1)
    accx = _agg128(x, srcm, dstm, z128)

    h1 = _tc_layer(accx[0, :N], accx[1, :N], deg, x, c1_Wl, c1_Wr,
                   c1_b.reshape(1, D))
    acc1 = _agg128(h1, srcm, dstm, z128)
    h2 = _tc_layer(acc1[0, :N], acc1[1, :N], deg, h1, c2_Wl, c2_Wr,
                   c2_b.reshape(1, D))
    acc2 = _agg128(h2, srcm, dstm, z128)

    Wlcat = _pad_cat([ca_Wl, cs_Wl, ce_Wl])
    h3, p3 = _tc_layer3(acc2[0, :N], acc2[1, :N], deg, h2, c3_Wl, c3_Wr,
                        c3_b.reshape(1, D), Wlcat)
    accp = _agg32(p3, srcm, dstm, z32)

    Wrcat = _pad_cat([ca_Wr, cs_Wr, ce_Wr])
    bcat = jnp.concatenate([ca_b, cs_b, ce_b,
                            jnp.zeros((HCAT - 28,), jnp.float32)]).reshape(1, HCAT)
    outh = _tc_heads(accp[0, :N], accp[1, :N], deg, h3, Wrcat, bcat)
    return outh[:, :21], outh[:, 21:23], outh[:, 23:28]


# speedup vs baseline: 1.9330x; 1.0444x over previous
import functools

import jax
import jax.numpy as jnp
from jax import lax
from jax.experimental import pallas as pl
from jax.experimental.pallas import tpu as pltpu
from jax.experimental.pallas import tpu_sc as plsc

N = 10000
D = 128
E = 320000

NC = 2
NS = 16
NW = NC * NS
CHUNK = 128
NCH0 = 96
NCH1 = 61
NCHMAX = 96
CH0 = NS * NCH0
CH_TOT = NS * (NCH0 + NCH1)
ROWS_TOT = 2560
EPAD = ROWS_TOT * CHUNK
SINK = N
AROWS = 10112
ZR = AROWS // NS

RB = 1000
DSIZE = 10240
HCAT = 32


def _chunk_bounds(c, s):
    b = jnp.where(c == 0, NCH0, NCH1)
    off = jnp.where(c == 0, s * NCH0, CH0 + s * NCH1)
    return b, off


def _make_agg(d):
    mesh = plsc.VectorSubcoreMesh(core_axis_name="c", subcore_axis_name="s",
                                  num_cores=NC, num_subcores=NS)
    out_type = jax.ShapeDtypeStruct((NC, AROWS, d), jnp.float32)
    scratch = [
        pltpu.VMEM((NCHMAX, CHUNK), jnp.int32),
        pltpu.VMEM((NCHMAX, CHUNK), jnp.int32),
        pltpu.VMEM((CHUNK, d), jnp.float32),
        pltpu.VMEM_SHARED((AROWS, d), jnp.float32),
        pltpu.SemaphoreType.DMA,
    ]

    def body(h, srcm, dstm, zrows, acc_out,
             src_v, dst_v, rows_v, acc_sh, sem):
        c = lax.axis_index("c")
        s = lax.axis_index("s")
        b, off = _chunk_bounds(c, s)
        pltpu.sync_copy(srcm.at[pl.ds(off, NCHMAX)], src_v)
        pltpu.sync_copy(dstm.at[pl.ds(off, NCHMAX)], dst_v)
        pltpu.sync_copy(zrows.at[pl.ds(s * ZR, ZR)], acc_sh.at[pl.ds(s * ZR, ZR)])
        plsc.subcore_barrier()

        def step(j, carry):
            @pl.when(j < b)
            def _():
                pltpu.async_copy(h.at[src_v.at[j]], rows_v, sem).wait()
                pltpu.sync_copy(rows_v, acc_sh.at[dst_v.at[j]], add=True)
            return carry

        lax.fori_loop(0, NCHMAX, step, 0)
        plsc.subcore_barrier()
        pltpu.sync_copy(acc_sh.at[pl.ds(s * ZR, ZR)],
                        acc_out.at[c, pl.ds(s * ZR, ZR)])

    return pl.kernel(body, out_type=out_type, mesh=mesh, scratch_types=scratch,
                     compiler_params=pltpu.CompilerParams(needs_layout_passes=False,
                                                          use_tc_tiling_on_sc=False))


def _make_deg():
    mesh = plsc.VectorSubcoreMesh(core_axis_name="c", subcore_axis_name="s",
                                  num_cores=NC, num_subcores=NS)
    out_type = jax.ShapeDtypeStruct((NC, NS, DSIZE), jnp.float32)
    scratch = [
        pltpu.VMEM((NCHMAX, CHUNK), jnp.int32),
        pltpu.VMEM((DSIZE,), jnp.float32),
        pltpu.SemaphoreType.DMA,
    ]

    def body(dstm, deg_out, dst_v, deg_v, sem):
        c = lax.axis_index("c")
        s = lax.axis_index("s")
        b, off = _chunk_bounds(c, s)
        cp = pltpu.async_copy(dstm.at[pl.ds(off, NCHMAX)], dst_v, sem)

        def zstep(i, carry):
            deg_v[pl.ds(i * 16, 16)] = jnp.zeros((16,), jnp.float32)
            return carry
        lax.fori_loop(0, DSIZE // 16, zstep, 0)
        cp.wait()

        ones = jnp.full((16,), 1.0, jnp.float32)

        def step(j, carry):
            @pl.when(j < b)
            def _():
                for t in range(CHUNK // 16):
                    dvec = dst_v[j, pl.ds(t * 16, 16)]
                    plsc.addupdate_scatter(deg_v, [dvec], ones)
            return carry

        lax.fori_loop(0, NCHMAX, step, 0)
        pltpu.sync_copy(deg_v, deg_out.at[c, s])

    return pl.kernel(body, out_type=out_type, mesh=mesh, scratch_types=scratch,
                     compiler_params=pltpu.CompilerParams(needs_layout_passes=False,
                                                          use_tc_tiling_on_sc=False))


_make_agg = functools.lru_cache(None)(_make_agg)
_make_deg = functools.lru_cache(None)(_make_deg)


def _agg128(*args):
    return _make_agg(D)(*args)


def _agg32(*args):
    return _make_agg(HCAT)(*args)


def _layer_body(a0, a1, deg, x, Wl, Wr, b, out):
    rd = 1.0 / jnp.maximum(deg[...], 1.0)
    mean = (a0[...] + a1[...]) * rd
    h = (jnp.dot(mean, Wl[...], preferred_element_type=jnp.float32)
         + jnp.dot(x[...], Wr[...], preferred_element_type=jnp.float32)
         + b[...])
    out[...] = jnp.maximum(h, 0.0)


def _layer3_body(a0, a1, deg, x, Wl, Wr, b, Wlcat, out, outp):
    rd = 1.0 / jnp.maximum(deg[...], 1.0)
    mean = (a0[...] + a1[...]) * rd
    h = (jnp.dot(mean, Wl[...], preferred_element_type=jnp.float32)
         + jnp.dot(x[...], Wr[...], preferred_element_type=jnp.float32)
         + b[...])
    h = jnp.maximum(h, 0.0)
    out[...] = h
    outp[...] = jnp.dot(h, Wlcat[...], preferred_element_type=jnp.float32)


def _heads_body(a0, a1, deg, h3, Wrcat, bcat, out):
    rd = 1.0 / jnp.maximum(deg[...], 1.0)
    meanp = (a0[...] + a1[...]) * rd
    out[...] = (meanp
                + jnp.dot(h3[...], Wrcat[...], preferred_element_type=jnp.float32)
                + bcat[...])


def _row_spec(cols):
    return pl.BlockSpec((RB, cols), lambda i: (i, 0))


def _full_spec(rows, cols):
    return pl.BlockSpec((rows, cols), lambda i: (0, 0))


def _tc_layer(a0, a1, deg, x, Wl, Wr, b):
    return pl.pallas_call(
        _layer_body,
        grid=(N // RB,),
        in_specs=[_row_spec(D), _row_spec(D), _row_spec(1), _row_spec(D),
                  _full_spec(D, D), _full_spec(D, D), _full_spec(1, D)],
        out_specs=_row_spec(D),
        out_shape=jax.ShapeDtypeStruct((N, D), jnp.float32),
    )(a0, a1, deg, x, Wl, Wr, b)


def _tc_layer3(a0, a1, deg, x, Wl, Wr, b, Wlcat):
    return pl.pallas_call(
        _layer3_body,
        grid=(N // RB,),
        in_specs=[_row_spec(D), _row_spec(D), _row_spec(1), _row_spec(D),
                  _full_spec(D, D), _full_spec(D, D), _full_spec(1, D),
                  _full_spec(D, HCAT)],
        out_specs=[_row_spec(D), _row_spec(HCAT)],
        out_shape=[jax.ShapeDtypeStruct((N, D), jnp.float32),
                   jax.ShapeDtypeStruct((N, HCAT), jnp.float32)],
    )(a0, a1, deg, x, Wl, Wr, b, Wlcat)


def _tc_heads(a0, a1, deg, h3, Wrcat, bcat):
    return pl.pallas_call(
        _heads_body,
        grid=(N // RB,),
        in_specs=[_row_spec(HCAT), _row_spec(HCAT), _row_spec(1), _row_spec(D),
                  _full_spec(D, HCAT), _full_spec(1, HCAT)],
        out_specs=_row_spec(HCAT),
        out_shape=jax.ShapeDtypeStruct((N, HCAT), jnp.float32),
    )(a0, a1, deg, h3, Wrcat, bcat)


def _pad_cat(ws):
    cat = jnp.concatenate(ws, axis=1)
    return jnp.pad(cat, ((0, 0), (0, HCAT - cat.shape[1])))


def kernel(x, edge_index, c1_Wl, c1_Wr, c1_b, c2_Wl, c2_Wr, c2_b,
           c3_Wl, c3_Wr, c3_b, ca_Wl, ca_Wr, ca_b, cs_Wl, cs_Wr, cs_b,
           ce_Wl, ce_Wr, ce_b):
    src = edge_index[0].astype(jnp.int32)
    dst = edge_index[1].astype(jnp.int32)
    pad = EPAD - E
    srcm = jnp.concatenate([src, jnp.zeros((pad,), jnp.int32)]).reshape(ROWS_TOT, CHUNK)
    sinks = SINK + (jnp.arange(pad, dtype=jnp.int32) % (AROWS - N))
    dstm = jnp.concatenate([dst, sinks]).reshape(ROWS_TOT, CHUNK)
    z128 = jnp.zeros((AROWS, D), jnp.float32)
    z32 = jnp.zeros((AROWS, HCAT), jnp.float32)

    degw = _make_deg()(dstm)
    deg = degw.reshape(NW, DSIZE).sum(axis=0)[:N].reshape(N, 1)
    accx = _agg128(x, srcm, dstm, z128)

    h1 = _tc_layer(accx[0, :N], accx[1, :N], deg, x, c1_Wl, c1_Wr,
                   c1_b.reshape(1, D))
    acc1 = _agg128(h1, srcm, dstm, z128)
    h2 = _tc_layer(acc1[0, :N], acc1[1, :N], deg, h1, c2_Wl, c2_Wr,
                   c2_b.reshape(1, D))
    acc2 = _agg128(h2, srcm, dstm, z128)

    Wlcat = _pad_cat([ca_Wl, cs_Wl, ce_Wl])
    h3, p3 = _tc_layer3(acc2[0, :N], acc2[1, :N], deg, h2, c3_Wl, c3_Wr,
                        c3_b.reshape(1, D), Wlcat)
    accp = _agg32(p3, srcm, dstm, z32)

    Wrcat = _pad_cat([ca_Wr, cs_Wr, ce_Wr])
    bcat = jnp.concatenate([ca_b, cs_b, ce_b,
                            jnp.zeros((HCAT - 28,), jnp.float32)]).reshape(1, HCAT)
    outh = _tc_heads(accp[0, :N], accp[1, :N], deg, h3, Wrcat, bcat)
    return outh[:, :21], outh[:, 21:23], outh[:, 23:28]
